# Initial kernel scaffold; baseline (speedup 1.0000x reference)
import jax
import jax.numpy as jnp
from jax.experimental import pallas as pl


def kernel(x):
    raise NotImplementedError("write your pallas kernel here")


# speedup vs baseline: 2.2110x; 2.2110x over previous
import jax
import jax.numpy as jnp
from jax.experimental import pallas as pl
from jax.experimental.pallas import tpu as pltpu

_N_BINS = 256
_TAU = 0.01
_EPS = 1e-10
_LANE = 128
_SUB = 32

_LOG2E = 1.4426950408889634
_C = -_LOG2E / (2.0 * _TAU * _TAU * 255.0 * 255.0)


def _weights(t_row, iota_bins):
    d = t_row - iota_bins
    return jnp.exp2((d * d) * _C)


def _hist_kernel(x_ref, hist_ref):
    i = pl.program_id(1)

    @pl.when(i == 0)
    def _():
        hist_ref[...] = jnp.zeros_like(hist_ref)

    t = x_ref[0] * 255.0
    iota_bins = jax.lax.broadcasted_iota(
        jnp.int32, (_N_BINS, _LANE), 0).astype(jnp.float32)
    acc = hist_ref[0]
    for k in range(_SUB):
        acc = acc + _weights(t[k : k + 1, :], iota_bins)
    hist_ref[0] = acc


def _eq_kernel(x_ref, hist_ref, out_ref, lhs_ref):
    i = pl.program_id(1)

    @pl.when(i == 0)
    def _():
        h = hist_ref[0]
        ones_row = jnp.ones((1, _LANE), jnp.float32)
        h_row = jax.lax.dot_general(
            ones_row, h, (((1,), (1,)), ((), ())),
            preferred_element_type=jnp.float32)
        r = jax.lax.broadcasted_iota(jnp.int32, (_N_BINS, _N_BINS), 0)
        c = jax.lax.broadcasted_iota(jnp.int32, (_N_BINS, _N_BINS), 1)
        tri = (r <= c).astype(jnp.float32)
        cdf = jnp.dot(h_row, tri, preferred_element_type=jnp.float32)
        total = cdf[:, _N_BINS - 1 :]
        cdf = cdf * (1.0 / (total + _EPS))
        c0 = cdf[:, 0:1]
        cdf_n = (cdf - c0) * (1.0 / (1.0 - c0 + _EPS))
        lhs = jnp.concatenate([jnp.ones_like(cdf_n), cdf_n], axis=0)
        lhs_ref[...] = lhs.astype(jnp.bfloat16)

    t = x_ref[0] * 255.0
    iota_bins = jax.lax.broadcasted_iota(
        jnp.int32, (_N_BINS, _LANE), 0).astype(jnp.float32)
    lhs = lhs_ref[...]
    for k in range(_SUB):
        w = _weights(t[k : k + 1, :], iota_bins).astype(jnp.bfloat16)
        dn = jnp.dot(lhs, w, preferred_element_type=jnp.float32)
        out_ref[0, k : k + 1, :] = dn[1:2, :] / (dn[0:1, :] + _EPS)


def kernel(x):
    B, _, H, W = x.shape
    hw_rows = (H * W) // _LANE
    nc = hw_rows // _SUB
    x3 = x.reshape(B, hw_rows, _LANE)

    hist = pl.pallas_call(
        _hist_kernel,
        grid=(B, nc),
        in_specs=[pl.BlockSpec((1, _SUB, _LANE), lambda b, i: (b, i, 0))],
        out_specs=pl.BlockSpec((1, _N_BINS, _LANE), lambda b, i: (b, 0, 0)),
        out_shape=jax.ShapeDtypeStruct((B, _N_BINS, _LANE), jnp.float32),
        compiler_params=pltpu.CompilerParams(
            dimension_semantics=("parallel", "arbitrary")),
    )(x3)

    out = pl.pallas_call(
        _eq_kernel,
        grid=(B, nc),
        in_specs=[
            pl.BlockSpec((1, _SUB, _LANE), lambda b, i: (b, i, 0)),
            pl.BlockSpec((1, _N_BINS, _LANE), lambda b, i: (b, 0, 0)),
        ],
        out_specs=pl.BlockSpec((1, _SUB, _LANE), lambda b, i: (b, i, 0)),
        out_shape=jax.ShapeDtypeStruct((B, hw_rows, _LANE), jnp.float32),
        scratch_shapes=[pltpu.VMEM((2, _N_BINS), jnp.bfloat16)],
        compiler_params=pltpu.CompilerParams(
            dimension_semantics=("parallel", "arbitrary")),
    )(x3, hist)

    return out.reshape(B, 1, H, W)
